# baseline (speedup 1.0000x reference)
import functools
import math

import jax
import jax.numpy as jnp
from jax.experimental import pallas as pl
from jax.experimental.pallas import tpu as pltpu
from jax.experimental.pallas import tpu_sc as plsc

D_MODEL = 32
NHEAD = 2
FF = 2048
CONV_CH = 24
GR = 12
KNN_K = 16
NUM_CONVS = 4
BB = 16
NN = 1024

NB_DENSE = 8
RB_EDGE = 1024
FF_CHUNK = 512
GD = 16

NUM_SC_WORKERS = 32
GATHER_ROWS = BB * NN * KNN_K
ROWS_PER_W = GATHER_ROWS // NUM_SC_WORKERS
SC_CHUNK = 2048

F32 = jnp.float32


def _bmm(a, b):
    return jax.lax.dot_general(
        a.astype(jnp.bfloat16), b.astype(jnp.bfloat16),
        (((a.ndim - 1,), (0,)), ((), ())),
        preferred_element_type=F32)


def _bmmT(a, w):
    return jax.lax.dot_general(
        a.astype(jnp.bfloat16), w.astype(jnp.bfloat16),
        (((a.ndim - 1,), (w.ndim - 1,)), ((), ())),
        preferred_element_type=F32)


def _layer_norm(x, w, b):
    m = jnp.mean(x, axis=-1, keepdims=True)
    v = jnp.mean((x - m) ** 2, axis=-1, keepdims=True)
    return (x - m) / jnp.sqrt(v + 1e-5) * w + b


def _dense_body(relu_in, xb_ref, pos_ref, mask_ref,
                wt_ref, bt_ref, wfc_ref, bfc_ref, wi_ref, bi_ref,
                wo_ref, bo_ref, w1_ref, b1_ref, w2_ref, b2_ref,
                n1w_ref, n1b_ref, n2w_ref, n2b_ref,
                wa1_ref, ba1_ref, wa2_ref, ba2_ref, wout_ref, bout_ref,
                wk1_ref, o_ref, g_ref):
    bb, nb = xb_ref.shape[0], xb_ref.shape[1]
    t_rows = bb * nb
    x = xb_ref[...].reshape(t_rows, xb_ref.shape[2])
    t = _bmmT(x, wt_ref[...]) + bt_ref[...]
    if relu_in:
        t = jnp.maximum(t, 0.0)
    h = _bmmT(t, wfc_ref[...]) + bfc_ref[...]
    h = (h.reshape(bb, nb, D_MODEL) + pos_ref[...][None, :, :]).reshape(t_rows, D_MODEL)

    qkv = _bmmT(h, wi_ref[...]) + bi_ref[...]
    dh = D_MODEL // NHEAD
    scale = 1.0 / math.sqrt(float(dh))
    mask = mask_ref[...]
    heads = []
    for hd in range(NHEAD):
        qh = qkv[:, hd * dh:(hd + 1) * dh]
        kh = qkv[:, D_MODEL + hd * dh:D_MODEL + (hd + 1) * dh]
        vh = qkv[:, 2 * D_MODEL + hd * dh:2 * D_MODEL + (hd + 1) * dh]
        s = _bmmT(qh, kh) * scale + mask
        s = s - jnp.max(s, axis=-1, keepdims=True)
        e = jnp.exp(s)
        a = e / jnp.sum(e, axis=-1, keepdims=True)
        heads.append(_bmm(a, vh))
    att = jnp.concatenate(heads, axis=-1)
    h = h + _bmmT(att, wo_ref[...]) + bo_ref[...]
    h = _layer_norm(h, n1w_ref[...], n1b_ref[...])

    acc = jnp.zeros((t_rows, D_MODEL), F32)
    for c in range(FF // FF_CHUNK):
        w1c = w1_ref[pl.ds(c * FF_CHUNK, FF_CHUNK), :]
        b1c = b1_ref[:, pl.ds(c * FF_CHUNK, FF_CHUNK)]
        hc = jnp.maximum(_bmmT(h, w1c) + b1c, 0.0)
        w2c = w2_ref[:, pl.ds(c * FF_CHUNK, FF_CHUNK)]
        acc = acc + _bmmT(hc, w2c)
    h = _layer_norm(h + acc + b2_ref[...], n2w_ref[...], n2b_ref[...])

    a1 = jnp.maximum(_bmmT(h, wa1_ref[...]) + ba1_ref[...], 0.0)
    a1b = a1.astype(jnp.bfloat16).astype(F32)
    wa2b = wa2_ref[...].astype(jnp.bfloat16).astype(F32)
    a2 = jnp.sum(a1b * wa2b, axis=1, keepdims=True) + ba2_ref[...]
    aw = 1.0 / (1.0 + jnp.exp(-a2))
    h = h * aw
    out = _bmmT(h, wout_ref[...]) + bout_ref[...]
    o_ref[...] = out.reshape(bb, nb, CONV_CH)
    g_ref[...] = _bmm(out, wk1_ref[...]).reshape(bb, nb, GD)


def _topk_body(frow_ref, ffull_ref, idx_ref):
    xt = frow_ref[0]
    f = ffull_ref[0]
    r = xt.shape[0]
    b = pl.program_id(0)

    sq_r = jnp.sum(xt * xt, axis=1, keepdims=True)
    ff2 = f * f
    sq_c = jax.lax.dot_general(
        jnp.ones((8, CONV_CH), F32), ff2,
        (((1,), (1,)), ((), ())),
        precision=jax.lax.Precision.HIGHEST,
        preferred_element_type=F32)[0:1, :]
    cross = _bmmT(xt, f)
    d = sq_r + sq_c - 2.0 * cross

    iota = jax.lax.broadcasted_iota(jnp.int32, (r, NN), 1)
    cols = []
    for j in range(KNN_K + 1):
        m = jnp.min(d, axis=1, keepdims=True)
        ismin = d <= m
        amin = jnp.min(jnp.where(ismin, iota, jnp.int32(2 ** 30)),
                       axis=1, keepdims=True)
        d = jnp.where(iota == amin, jnp.float32(jnp.inf), d)
        if j > 0:
            cols.append(amin)
    idx_ref[0] = jnp.concatenate(cols, axis=1) + b * NN


def _edge_body(frow_ref, gath_ref,
               wx1_ref, b1_ref, wm_ref, wx2_ref, b2_ref,
               wlm_ref, wlf_ref, wx3_ref, b3_ref,
               o_ref):
    xt = frow_ref[0]
    r = xt.shape[0]

    cx1 = _bmm(xt, wx1_ref[...]) + b1_ref[...]
    cx2 = _bmm(xt, wx2_ref[...]) + b2_ref[...]
    cx3 = _bmm(xt, wx3_ref[...]) + b3_ref[...]

    neg = jnp.float32(-jnp.inf)
    acc_l = jnp.full((r, GR), neg, F32)
    acc_m = jnp.full((r, GR), neg, F32)
    acc_f = jnp.full((r, GR), neg, F32)
    for j in range(KNN_K):
        gj = gath_ref[0][:, j * GD:j * GD + GR]
        first = jnp.maximum(gj + cx1, 0.0)
        mid = jnp.maximum(_bmm(first, wm_ref[...]) + cx2, 0.0)
        last = _bmm(mid, wlm_ref[...]) + _bmm(first, wlf_ref[...]) + cx3
        acc_l = jnp.maximum(acc_l, last)
        acc_m = jnp.maximum(acc_m, mid)
        acc_f = jnp.maximum(acc_f, first)
    o_ref[0] = jnp.concatenate([acc_l, acc_m, acc_f, xt], axis=1)


def _sc_gather(table, idx):
    mesh = plsc.VectorSubcoreMesh(core_axis_name="c", subcore_axis_name="s")

    @functools.partial(
        pl.kernel, mesh=mesh,
        compiler_params=pltpu.CompilerParams(use_tc_tiling_on_sc=False),
        out_type=jax.ShapeDtypeStruct((GATHER_ROWS, GD), F32),
        scratch_types=[
            pltpu.VMEM((SC_CHUNK,), jnp.int32),
            pltpu.VMEM((SC_CHUNK, GD), F32),
            pltpu.SemaphoreType.DMA,
        ],
    )
    def k(table_hbm, idx_hbm, out_hbm, idx_v, rows_v, sem):
        wid = jax.lax.axis_index("s") * 2 + jax.lax.axis_index("c")
        base = wid * ROWS_PER_W

        def body(c, _):
            off = base + c * SC_CHUNK
            pltpu.sync_copy(idx_hbm.at[pl.ds(off, SC_CHUNK)], idx_v)
            pltpu.async_copy(table_hbm.at[idx_v], rows_v, sem).wait()
            pltpu.sync_copy(rows_v, out_hbm.at[pl.ds(off, SC_CHUNK)])
            return _

        jax.lax.fori_loop(0, ROWS_PER_W // SC_CHUNK, body, 0)

    return k(table, idx)


def _edge_weights(p, i):
    w1, b1 = p['conv%d_first' % i]
    w2, b2 = p['conv%d_mid0' % i]
    w3, b3 = p['conv%d_last' % i]
    if i == 0:
        wk1 = w1.T
        wx1 = -w1.T
    else:
        a, bw, cw = w1[:, :CONV_CH], w1[:, CONV_CH:2 * CONV_CH], w1[:, 2 * CONV_CH:]
        wk1 = (bw + cw).T
        wx1 = (a - cw).T
    wk1 = jnp.pad(wk1, ((0, 0), (0, GD - GR)))
    wm = w2[:, :GR].T
    wx2 = w2[:, GR:].T
    wlm = w3[:, :GR].T
    wlf = w3[:, GR:2 * GR].T
    wx3 = w3[:, 2 * GR:].T
    r2 = lambda v: v.reshape(1, -1)
    return wk1, (wx1, r2(b1), wm, wx2, r2(b2), wlm, wlf, wx3, r2(b3))


def _dense_call(x_t, p, i, wk1):
    in_ch = x_t.shape[2]
    wt, bt = p['trans%d' % i]
    wfc, bfc = p['nt_fc_in']
    wi, bi = p['nt_in_proj']
    wo, bo = p['nt_out_proj']
    w1, b1 = p['nt_lin1']
    w2, b2 = p['nt_lin2']
    n1w, n1b = p['nt_norm1']
    n2w, n2b = p['nt_norm2']
    wa1, ba1 = p['nt_attn1']
    wa2, ba2 = p['nt_attn2']
    wout, bout = p['nt_fc_out']
    pos = p['nt_pos'][0]
    r2 = lambda v: v.reshape(1, -1)

    t_rows = NB_DENSE * BB
    gid = jnp.arange(t_rows) % NB_DENSE
    mask = jnp.where(gid[:, None] == gid[None, :], 0.0, -1e30).astype(F32)

    full = lambda a: pl.BlockSpec(a.shape, lambda n: (0,) * a.ndim)
    args = (x_t, pos, mask,
            wt, r2(bt), wfc, r2(bfc), wi, r2(bi), wo, r2(bo),
            w1, r2(b1), w2, r2(b2), r2(n1w), r2(n1b), r2(n2w), r2(n2b),
            wa1, r2(ba1), wa2, r2(ba2), wout, r2(bout), wk1)
    in_specs = [
        pl.BlockSpec((BB, NB_DENSE, in_ch), lambda n: (0, n, 0)),
        pl.BlockSpec((NB_DENSE, D_MODEL), lambda n: (n, 0)),
    ] + [full(a) for a in args[2:]]
    return pl.pallas_call(
        functools.partial(_dense_body, i > 0),
        grid=(NN // NB_DENSE,),
        in_specs=in_specs,
        out_specs=[
            pl.BlockSpec((BB, NB_DENSE, CONV_CH), lambda n: (0, n, 0)),
            pl.BlockSpec((BB, NB_DENSE, GD), lambda n: (0, n, 0)),
        ],
        out_shape=[
            jax.ShapeDtypeStruct((BB, NN, CONV_CH), F32),
            jax.ShapeDtypeStruct((BB, NN, GD), F32),
        ],
    )(*args)


def _topk_call(feat_bn):
    return pl.pallas_call(
        _topk_body,
        grid=(BB, NN // RB_EDGE),
        in_specs=[
            pl.BlockSpec((1, RB_EDGE, CONV_CH), lambda b, r: (b, r, 0)),
            pl.BlockSpec((1, NN, CONV_CH), lambda b, r: (b, 0, 0)),
        ],
        out_specs=pl.BlockSpec((1, RB_EDGE, KNN_K), lambda b, r: (b, r, 0)),
        out_shape=jax.ShapeDtypeStruct((BB, NN, KNN_K), jnp.int32),
    )(feat_bn, feat_bn)


def _edge_call(feat_bn, gath_bn, ws):
    full = lambda a: pl.BlockSpec(a.shape, lambda b, r: (0,) * a.ndim)
    in_specs = [
        pl.BlockSpec((1, RB_EDGE, CONV_CH), lambda b, r: (b, r, 0)),
        pl.BlockSpec((1, RB_EDGE, KNN_K * GD), lambda b, r: (b, r, 0)),
    ] + [full(a) for a in ws]
    out_ch = CONV_CH + 3 * GR
    return pl.pallas_call(
        _edge_body,
        grid=(BB, NN // RB_EDGE),
        in_specs=in_specs,
        out_specs=pl.BlockSpec((1, RB_EDGE, out_ch), lambda b, r: (b, r, 0)),
        out_shape=jax.ShapeDtypeStruct((BB, NN, out_ch), F32),
    )(feat_bn, gath_bn, *ws)


def kernel(x, params):
    for i in range(NUM_CONVS):
        wk1, ws = _edge_weights(params, i)
        feat_bn, g_bn = _dense_call(x, params, i, wk1)
        g_flat = g_bn.reshape(BB * NN, GD)
        idx = _topk_call(feat_bn)
        gath = _sc_gather(g_flat, idx.reshape(GATHER_ROWS))
        gath_bn = gath.reshape(BB, NN, KNN_K * GD)
        x = _edge_call(feat_bn, gath_bn, ws)
    return x


# speedup vs baseline: 1.3402x; 1.3402x over previous
import functools
import math

import jax
import jax.numpy as jnp
from jax.experimental import pallas as pl
from jax.experimental.pallas import tpu as pltpu
from jax.experimental.pallas import tpu_sc as plsc

D_MODEL = 32
NHEAD = 2
FF = 2048
CONV_CH = 24
GR = 12
KNN_K = 16
NUM_CONVS = 4
BB = 16
NN = 1024

NB_DENSE = 32
RB_EDGE = 1024
FF_CHUNK = 512
GD = 16

NUM_SC_WORKERS = 32
GATHER_ROWS = BB * NN * KNN_K
ROWS_PER_W = GATHER_ROWS // NUM_SC_WORKERS
SC_CHUNK = 2048

F32 = jnp.float32


def _bmm(a, b):
    return jax.lax.dot_general(
        a.astype(jnp.bfloat16), b.astype(jnp.bfloat16),
        (((a.ndim - 1,), (0,)), ((), ())),
        preferred_element_type=F32)


def _bmmT(a, w):
    return jax.lax.dot_general(
        a.astype(jnp.bfloat16), w.astype(jnp.bfloat16),
        (((a.ndim - 1,), (w.ndim - 1,)), ((), ())),
        preferred_element_type=F32)


def _layer_norm(x, w, b):
    m = jnp.mean(x, axis=-1, keepdims=True)
    v = jnp.mean((x - m) ** 2, axis=-1, keepdims=True)
    return (x - m) / jnp.sqrt(v + 1e-5) * w + b


def _dense_body(relu_in, xb_ref, pos_ref, mask_ref,
                wt_ref, bt_ref, wfc_ref, bfc_ref, wi_ref, bi_ref,
                wo_ref, bo_ref, w1_ref, b1_ref, w2_ref, b2_ref,
                n1w_ref, n1b_ref, n2w_ref, n2b_ref,
                wa1_ref, ba1_ref, wa2_ref, ba2_ref, wout_ref, bout_ref,
                wk1_ref, o_ref, g_ref):
    bb, nb = xb_ref.shape[0], xb_ref.shape[1]
    t_rows = bb * nb
    x = xb_ref[...].reshape(t_rows, xb_ref.shape[2])
    t = _bmmT(x, wt_ref[...]) + bt_ref[...]
    if relu_in:
        t = jnp.maximum(t, 0.0)
    h = _bmmT(t, wfc_ref[...]) + bfc_ref[...]
    h = (h.reshape(bb, nb, D_MODEL) + pos_ref[...][None, :, :]).reshape(t_rows, D_MODEL)

    qkv = _bmmT(h, wi_ref[...]) + bi_ref[...]
    dh = D_MODEL // NHEAD
    scale = 1.0 / math.sqrt(float(dh))
    mask = mask_ref[...]
    heads = []
    for hd in range(NHEAD):
        qh = qkv[:, hd * dh:(hd + 1) * dh]
        kh = qkv[:, D_MODEL + hd * dh:D_MODEL + (hd + 1) * dh]
        vh = qkv[:, 2 * D_MODEL + hd * dh:2 * D_MODEL + (hd + 1) * dh]
        s = _bmmT(qh, kh) * scale + mask
        s = s - jnp.max(s, axis=-1, keepdims=True)
        e = jnp.exp(s)
        a = e / jnp.sum(e, axis=-1, keepdims=True)
        heads.append(_bmm(a, vh))
    att = jnp.concatenate(heads, axis=-1)
    h = h + _bmmT(att, wo_ref[...]) + bo_ref[...]
    h = _layer_norm(h, n1w_ref[...], n1b_ref[...])

    acc = jnp.zeros((t_rows, D_MODEL), F32)
    for c in range(FF // FF_CHUNK):
        w1c = w1_ref[pl.ds(c * FF_CHUNK, FF_CHUNK), :]
        b1c = b1_ref[:, pl.ds(c * FF_CHUNK, FF_CHUNK)]
        hc = jnp.maximum(_bmmT(h, w1c) + b1c, 0.0)
        w2c = w2_ref[:, pl.ds(c * FF_CHUNK, FF_CHUNK)]
        acc = acc + _bmmT(hc, w2c)
    h = _layer_norm(h + acc + b2_ref[...], n2w_ref[...], n2b_ref[...])

    a1 = jnp.maximum(_bmmT(h, wa1_ref[...]) + ba1_ref[...], 0.0)
    a1b = a1.astype(jnp.bfloat16).astype(F32)
    wa2b = wa2_ref[...].astype(jnp.bfloat16).astype(F32)
    a2 = jnp.sum(a1b * wa2b, axis=1, keepdims=True) + ba2_ref[...]
    aw = 1.0 / (1.0 + jnp.exp(-a2))
    h = h * aw
    out = _bmmT(h, wout_ref[...]) + bout_ref[...]
    o_ref[...] = out.reshape(bb, nb, CONV_CH)
    g_ref[...] = _bmm(out, wk1_ref[...]).reshape(bb, nb, GD)


def _topk_body(frow_ref, ffull_ref, idx_ref):
    xt = frow_ref[0]
    f = ffull_ref[0]
    r = xt.shape[0]
    b = pl.program_id(0)

    sq_r = jnp.sum(xt * xt, axis=1, keepdims=True)
    ff2 = f * f
    sq_c = jax.lax.dot_general(
        jnp.ones((8, CONV_CH), F32), ff2,
        (((1,), (1,)), ((), ())),
        precision=jax.lax.Precision.HIGHEST,
        preferred_element_type=F32)[0:1, :]
    cross = _bmmT(xt, f)
    d = sq_r + sq_c - 2.0 * cross

    iota = jax.lax.broadcasted_iota(jnp.int32, (r, NN), 1)
    cols = []
    for j in range(KNN_K + 1):
        m = jnp.min(d, axis=1, keepdims=True)
        ismin = d <= m
        amin = jnp.min(jnp.where(ismin, iota, jnp.int32(2 ** 30)),
                       axis=1, keepdims=True)
        d = jnp.where(iota == amin, jnp.float32(jnp.inf), d)
        if j > 0:
            cols.append(amin)
    idx_ref[0] = jnp.concatenate(cols, axis=1) + b * NN


def _edge_body(frow_ref, gath_ref,
               wx1_ref, b1_ref, wm_ref, wx2_ref, b2_ref,
               wlm_ref, wlf_ref, wx3_ref, b3_ref,
               o_ref):
    xt = frow_ref[0]
    r = xt.shape[0]

    cx1 = _bmm(xt, wx1_ref[...]) + b1_ref[...]
    cx2 = _bmm(xt, wx2_ref[...]) + b2_ref[...]
    cx3 = _bmm(xt, wx3_ref[...]) + b3_ref[...]

    neg = jnp.float32(-jnp.inf)
    acc_l = jnp.full((r, GR), neg, F32)
    acc_m = jnp.full((r, GR), neg, F32)
    acc_f = jnp.full((r, GR), neg, F32)
    for j in range(KNN_K):
        gj = gath_ref[0][:, j * GD:j * GD + GR]
        first = jnp.maximum(gj + cx1, 0.0)
        mid = jnp.maximum(_bmm(first, wm_ref[...]) + cx2, 0.0)
        last = _bmm(mid, wlm_ref[...]) + _bmm(first, wlf_ref[...]) + cx3
        acc_l = jnp.maximum(acc_l, last)
        acc_m = jnp.maximum(acc_m, mid)
        acc_f = jnp.maximum(acc_f, first)
    o_ref[0] = jnp.concatenate([acc_l, acc_m, acc_f, xt], axis=1)


def _sc_gather(table, idx):
    mesh = plsc.VectorSubcoreMesh(core_axis_name="c", subcore_axis_name="s")

    @functools.partial(
        pl.kernel, mesh=mesh,
        compiler_params=pltpu.CompilerParams(use_tc_tiling_on_sc=False),
        out_type=jax.ShapeDtypeStruct((GATHER_ROWS, GD), F32),
        scratch_types=[
            pltpu.VMEM((SC_CHUNK,), jnp.int32),
            pltpu.VMEM((SC_CHUNK, GD), F32),
            pltpu.SemaphoreType.DMA,
        ],
    )
    def k(table_hbm, idx_hbm, out_hbm, idx_v, rows_v, sem):
        wid = jax.lax.axis_index("s") * 2 + jax.lax.axis_index("c")
        base = wid * ROWS_PER_W

        def body(c, _):
            off = base + c * SC_CHUNK
            pltpu.sync_copy(idx_hbm.at[pl.ds(off, SC_CHUNK)], idx_v)
            pltpu.async_copy(table_hbm.at[idx_v], rows_v, sem).wait()
            pltpu.sync_copy(rows_v, out_hbm.at[pl.ds(off, SC_CHUNK)])
            return _

        jax.lax.fori_loop(0, ROWS_PER_W // SC_CHUNK, body, 0)

    return k(table, idx)


def _edge_weights(p, i):
    w1, b1 = p['conv%d_first' % i]
    w2, b2 = p['conv%d_mid0' % i]
    w3, b3 = p['conv%d_last' % i]
    if i == 0:
        wk1 = w1.T
        wx1 = -w1.T
    else:
        a, bw, cw = w1[:, :CONV_CH], w1[:, CONV_CH:2 * CONV_CH], w1[:, 2 * CONV_CH:]
        wk1 = (bw + cw).T
        wx1 = (a - cw).T
    wk1 = jnp.pad(wk1, ((0, 0), (0, GD - GR)))
    wm = w2[:, :GR].T
    wx2 = w2[:, GR:].T
    wlm = w3[:, :GR].T
    wlf = w3[:, GR:2 * GR].T
    wx3 = w3[:, 2 * GR:].T
    r2 = lambda v: v.reshape(1, -1)
    return wk1, (wx1, r2(b1), wm, wx2, r2(b2), wlm, wlf, wx3, r2(b3))


def _dense_call(x_t, p, i, wk1):
    in_ch = x_t.shape[2]
    wt, bt = p['trans%d' % i]
    wfc, bfc = p['nt_fc_in']
    wi, bi = p['nt_in_proj']
    wo, bo = p['nt_out_proj']
    w1, b1 = p['nt_lin1']
    w2, b2 = p['nt_lin2']
    n1w, n1b = p['nt_norm1']
    n2w, n2b = p['nt_norm2']
    wa1, ba1 = p['nt_attn1']
    wa2, ba2 = p['nt_attn2']
    wout, bout = p['nt_fc_out']
    pos = p['nt_pos'][0]
    r2 = lambda v: v.reshape(1, -1)

    t_rows = NB_DENSE * BB
    gid = jnp.arange(t_rows) % NB_DENSE
    mask = jnp.where(gid[:, None] == gid[None, :], 0.0, -1e30).astype(F32)

    full = lambda a: pl.BlockSpec(a.shape, lambda n: (0,) * a.ndim)
    args = (x_t, pos, mask,
            wt, r2(bt), wfc, r2(bfc), wi, r2(bi), wo, r2(bo),
            w1, r2(b1), w2, r2(b2), r2(n1w), r2(n1b), r2(n2w), r2(n2b),
            wa1, r2(ba1), wa2, r2(ba2), wout, r2(bout), wk1)
    in_specs = [
        pl.BlockSpec((BB, NB_DENSE, in_ch), lambda n: (0, n, 0)),
        pl.BlockSpec((NB_DENSE, D_MODEL), lambda n: (n, 0)),
    ] + [full(a) for a in args[2:]]
    return pl.pallas_call(
        functools.partial(_dense_body, i > 0),
        grid=(NN // NB_DENSE,),
        in_specs=in_specs,
        out_specs=[
            pl.BlockSpec((BB, NB_DENSE, CONV_CH), lambda n: (0, n, 0)),
            pl.BlockSpec((BB, NB_DENSE, GD), lambda n: (0, n, 0)),
        ],
        out_shape=[
            jax.ShapeDtypeStruct((BB, NN, CONV_CH), F32),
            jax.ShapeDtypeStruct((BB, NN, GD), F32),
        ],
    )(*args)


def _topk_call(feat_bn):
    return pl.pallas_call(
        _topk_body,
        grid=(BB, NN // RB_EDGE),
        in_specs=[
            pl.BlockSpec((1, RB_EDGE, CONV_CH), lambda b, r: (b, r, 0)),
            pl.BlockSpec((1, NN, CONV_CH), lambda b, r: (b, 0, 0)),
        ],
        out_specs=pl.BlockSpec((1, RB_EDGE, KNN_K), lambda b, r: (b, r, 0)),
        out_shape=jax.ShapeDtypeStruct((BB, NN, KNN_K), jnp.int32),
    )(feat_bn, feat_bn)


def _edge_call(feat_bn, gath_bn, ws):
    full = lambda a: pl.BlockSpec(a.shape, lambda b, r: (0,) * a.ndim)
    in_specs = [
        pl.BlockSpec((1, RB_EDGE, CONV_CH), lambda b, r: (b, r, 0)),
        pl.BlockSpec((1, RB_EDGE, KNN_K * GD), lambda b, r: (b, r, 0)),
    ] + [full(a) for a in ws]
    out_ch = CONV_CH + 3 * GR
    return pl.pallas_call(
        _edge_body,
        grid=(BB, NN // RB_EDGE),
        in_specs=in_specs,
        out_specs=pl.BlockSpec((1, RB_EDGE, out_ch), lambda b, r: (b, r, 0)),
        out_shape=jax.ShapeDtypeStruct((BB, NN, out_ch), F32),
    )(feat_bn, gath_bn, *ws)


def kernel(x, params):
    for i in range(NUM_CONVS):
        wk1, ws = _edge_weights(params, i)
        feat_bn, g_bn = _dense_call(x, params, i, wk1)
        g_flat = g_bn.reshape(BB * NN, GD)
        idx = _topk_call(feat_bn)
        gath = _sc_gather(g_flat, idx.reshape(GATHER_ROWS))
        gath_bn = gath.reshape(BB, NN, KNN_K * GD)
        x = _edge_call(feat_bn, gath_bn, ws)
    return x
